# baseline (speedup 1.0000x reference)
import functools

import jax
import jax.numpy as jnp
from jax import lax
from jax.experimental import pallas as pl
from jax.experimental.pallas import tpu as pltpu
from jax.experimental.pallas import tpu_sc as plsc

N = 16384
D = 768
P = 16
C = 2048
PC = P * C
NC = 2
NS = 16
NW = NC * NS
L = 16

TOK_W = N // NW
ROWS_W = PC // NW
HALF = ROWS_W
CHUNK = 64
NCHUNK = ROWS_W // CHUNK
NBUF = 2
ZROWS = 16

_mesh = plsc.VectorSubcoreMesh(core_axis_name="c", subcore_axis_name="s")
_params = pltpu.CompilerParams(needs_layout_passes=False)


def _wid():
    return lax.axis_index("s") * NC + lax.axis_index("c")


def _scalar(a):
    return jnp.max(a) if a.ndim else a


@functools.partial(
    pl.kernel,
    out_type=jax.ShapeDtypeStruct((N,), jnp.int32),
    mesh=_mesh,
    compiler_params=_params,
    scratch_types=[
        pltpu.VMEM((TOK_W, P), jnp.float32),
        pltpu.VMEM((TOK_W,), jnp.int32),
    ],
)
def _route(scores_hbm, packed_hbm, sbuf, obuf):
    base = _wid() * TOK_W
    pltpu.sync_copy(scores_hbm.at[pl.ds(base, TOK_W)], sbuf)
    iota = lax.iota(jnp.int32, L)

    def body(t0, carry):
        rows = iota + t0 * L
        m = plsc.load_gather(sbuf, [rows, jnp.zeros((L,), jnp.int32)])
        am = jnp.zeros((L,), jnp.int32)
        for p in range(1, P):
            v = plsc.load_gather(sbuf, [rows, jnp.full((L,), p, jnp.int32)])
            gt = v > m
            m = jnp.where(gt, v, m)
            am = jnp.where(gt, p, am)
        packed = (lax.bitcast_convert_type(m, jnp.int32) & -16) | am
        obuf[pl.ds(t0 * L, L)] = packed
        return carry

    lax.fori_loop(0, TOK_W // L, body, 0)
    pltpu.sync_copy(obuf, packed_hbm.at[pl.ds(base, TOK_W)])


@functools.partial(
    pl.kernel,
    out_type=(),
    mesh=_mesh,
    compiler_params=_params,
    scratch_types=[
        pltpu.VMEM((N,), jnp.int32),
        pltpu.VMEM((C + L,), jnp.int32),
        [pltpu.VMEM((CHUNK,), jnp.int32) for _ in range(NBUF)],
        [pltpu.VMEM((CHUNK,), jnp.float32) for _ in range(NBUF)],
        [pltpu.VMEM((CHUNK, D), jnp.float32) for _ in range(NBUF)],
        pltpu.VMEM((L,), jnp.int32),
        [pltpu.SemaphoreType.DMA for _ in range(NBUF)],
        [pltpu.SemaphoreType.DMA for _ in range(NBUF)],
    ],
)
def _dispatch(x_hbm, packed_hbm, out_hbm, counts_hbm,
              pvb, cand, cidx, gch, rowbuf, cbuf, gsem, wsem):
    w = _wid()
    pno = w // 2
    h = w % 2
    pltpu.sync_copy(packed_hbm, pvb)

    iota = lax.iota(jnp.int32, L)

    def scan(i, cnt_s):
        v = pvb[pl.ds(i * L, L)]
        msk = (v & 15) == pno
        plsc.store_compressed(cand.at[pl.ds(cnt_s, L)], iota + i * L, mask=msk)
        pc = plsc.all_reduce_population_count(msk)
        if pc.ndim:
            pc = pc[0]
        return jnp.minimum(cnt_s + pc, C)

    cnt = lax.fori_loop(0, N // L, scan, jnp.zeros((), jnp.int32))

    @pl.when(h == 0)
    def _publish_count():
        cbuf[...] = jnp.broadcast_to(cnt, (L,))
        pltpu.sync_copy(cbuf, counts_hbm.at[pno])

    path_base = pno * C
    nocc = jnp.clip((cnt - h * CHUNK + 2 * CHUNK - 1) // (2 * CHUNK), 0, NCHUNK)

    def rank_of(c):
        return (2 * c + h) * CHUNK

    def fill_gather(c, b):
        rank0 = rank_of(c)
        for u in range(CHUNK // L):
            r = iota + (rank0 + u * L)
            valid = r < cnt
            ids = cand[pl.ds(rank0 + u * L, L)]
            ids = jnp.where(valid, ids, 0)
            pk = plsc.load_gather(pvb, [ids])
            g = lax.bitcast_convert_type(pk & -16, jnp.float32)
            g = jnp.where(valid, g, 0.0)
            cidx[b][pl.ds(u * L, L)] = ids
            gch[b][pl.ds(u * L, L)] = g
        pltpu.async_copy(x_hbm.at[cidx[b]], rowbuf[b], gsem[b])

    def scale_write(c, b):
        pltpu.make_async_copy(x_hbm.at[cidx[b]], rowbuf[b], gsem[b]).wait()

        def srow(j, carry2):
            gs = plsc.load_gather(gch[b], [jnp.zeros((L,), jnp.int32) + j])
            for k in range(D // L):
                rowbuf[b][j, pl.ds(k * L, L)] = (
                    rowbuf[b][j, pl.ds(k * L, L)] * gs)
            return carry2

        lax.fori_loop(0, CHUNK, srow, 0)
        pltpu.async_copy(
            rowbuf[b], out_hbm.at[pl.ds(path_base + rank_of(c), CHUNK)],
            wsem[b])

    @pl.when(nocc > 0)
    def _prime():
        fill_gather(0, 0)

    def group_body(grp, carry):
        for b in range(NBUF):
            c = grp * NBUF + b

            @pl.when(c < nocc)
            def _step(c=c, b=b):
                b1 = (b + 1) % NBUF
                cn = c + 1

                @pl.when(cn < nocc)
                def _lookahead():
                    @pl.when(cn >= NBUF)
                    def _reuse_wait():
                        pltpu.make_async_copy(
                            rowbuf[b1],
                            out_hbm.at[pl.ds(path_base, CHUNK)],
                            wsem[b1]).wait()

                    fill_gather(cn, b1)

                scale_write(c, b)

        return carry

    lax.fori_loop(0, (NCHUNK + NBUF - 1) // NBUF, group_body, 0)

    for b in range(NBUF):
        used = jnp.zeros((), jnp.bool_)
        for k in range(1, NBUF + 1):
            used = used | ((nocc >= k) & (lax.rem(nocc - k, NBUF) == b))

        @pl.when(used)
        def _drain(b=b):
            pltpu.make_async_copy(
                rowbuf[b], out_hbm.at[pl.ds(path_base, CHUNK)],
                wsem[b]).wait()


ZBIG = 512
ZSM = CHUNK


@functools.partial(
    pl.kernel,
    out_type=(),
    mesh=pltpu.create_tensorcore_mesh("core", num_cores=1),
    scratch_types=[
        pltpu.VMEM((ZBIG, D), jnp.float32),
        pltpu.SMEM((P, L), jnp.int32),
        pltpu.SemaphoreType.DMA,
        pltpu.SemaphoreType.DMA,
    ],
)
def _tc_zero(counts_hbm, out_hbm, zbuf, csm, sem_b, sem_s):
    pltpu.sync_copy(counts_hbm, csm)
    zbuf[...] = jnp.zeros((ZBIG, D), jnp.float32)
    nb_total = jnp.zeros((), jnp.int32)
    ns_total = jnp.zeros((), jnp.int32)
    for p in range(P):
        cnt = csm[p, 0]
        zs = ((cnt + CHUNK - 1) // CHUNK) * CHUNK
        za = jnp.minimum(((zs + ZBIG - 1) // ZBIG) * ZBIG, C)
        ns = (za - zs) // ZSM
        nb = (C - za) // ZBIG
        base = p * C

        def sfire(i, carry, base=base, zs=zs):
            pltpu.async_copy(
                zbuf.at[pl.ds(0, ZSM)],
                out_hbm.at[pl.ds(base + zs + i * ZSM, ZSM)], sem_s)
            return carry

        lax.fori_loop(0, ns, sfire, 0)

        def bfire(i, carry, base=base, za=za):
            pltpu.async_copy(
                zbuf, out_hbm.at[pl.ds(base + za + i * ZBIG, ZBIG)], sem_b)
            return carry

        lax.fori_loop(0, nb, bfire, 0)
        nb_total = nb_total + nb
        ns_total = ns_total + ns

    def sdrain(i, carry):
        pltpu.make_async_copy(
            zbuf.at[pl.ds(0, ZSM)], out_hbm.at[pl.ds(0, ZSM)], sem_s).wait()
        return carry

    lax.fori_loop(0, ns_total, sdrain, 0)

    def bdrain(i, carry):
        pltpu.make_async_copy(
            zbuf, out_hbm.at[pl.ds(0, ZBIG)], sem_b).wait()
        return carry

    lax.fori_loop(0, nb_total, bdrain, 0)


def kernel(x, scores):
    packed = _route(scores)
    out_ref = jax.new_ref(lax.empty((PC, D), jnp.float32))
    counts_ref = jax.new_ref(lax.empty((P, L), jnp.int32))
    _dispatch(x, packed, out_ref, counts_ref)
    _tc_zero(counts_ref, out_ref)
    return out_ref[...]


# speedup vs baseline: 1.1160x; 1.1160x over previous
import functools

import jax
import jax.numpy as jnp
from jax import lax
from jax.experimental import pallas as pl
from jax.experimental.pallas import tpu as pltpu
from jax.experimental.pallas import tpu_sc as plsc

N = 16384
D = 768
P = 16
C = 2048
PC = P * C
NC = 2
NS = 16
NW = NC * NS
L = 16

TOK_W = N // NW
ROWS_W = PC // NW
HALF = ROWS_W
CHUNK = 32
NCHUNK = ROWS_W // CHUNK
NBUF = 4
ZROWS = 16

_mesh = plsc.VectorSubcoreMesh(core_axis_name="c", subcore_axis_name="s")
_params = pltpu.CompilerParams(needs_layout_passes=False)


def _wid():
    return lax.axis_index("s") * NC + lax.axis_index("c")


def _scalar(a):
    return jnp.max(a) if a.ndim else a


@functools.partial(
    pl.kernel,
    out_type=jax.ShapeDtypeStruct((N,), jnp.int32),
    mesh=_mesh,
    compiler_params=_params,
    scratch_types=[
        pltpu.VMEM((TOK_W, P), jnp.float32),
        pltpu.VMEM((TOK_W,), jnp.int32),
    ],
)
def _route(scores_hbm, packed_hbm, sbuf, obuf):
    base = _wid() * TOK_W
    pltpu.sync_copy(scores_hbm.at[pl.ds(base, TOK_W)], sbuf)
    iota = lax.iota(jnp.int32, L)

    def body(t0, carry):
        rows = iota + t0 * L
        m = plsc.load_gather(sbuf, [rows, jnp.zeros((L,), jnp.int32)])
        am = jnp.zeros((L,), jnp.int32)
        for p in range(1, P):
            v = plsc.load_gather(sbuf, [rows, jnp.full((L,), p, jnp.int32)])
            gt = v > m
            m = jnp.where(gt, v, m)
            am = jnp.where(gt, p, am)
        packed = (lax.bitcast_convert_type(m, jnp.int32) & -16) | am
        obuf[pl.ds(t0 * L, L)] = packed
        return carry

    lax.fori_loop(0, TOK_W // L, body, 0)
    pltpu.sync_copy(obuf, packed_hbm.at[pl.ds(base, TOK_W)])


@functools.partial(
    pl.kernel,
    out_type=(),
    mesh=_mesh,
    compiler_params=_params,
    scratch_types=[
        pltpu.VMEM((N,), jnp.int32),
        pltpu.VMEM((C + L,), jnp.int32),
        [pltpu.VMEM((CHUNK,), jnp.int32) for _ in range(NBUF)],
        [pltpu.VMEM((CHUNK,), jnp.float32) for _ in range(NBUF)],
        [pltpu.VMEM((CHUNK, D), jnp.float32) for _ in range(NBUF)],
        pltpu.VMEM((L,), jnp.int32),
        [pltpu.SemaphoreType.DMA for _ in range(NBUF)],
        [pltpu.SemaphoreType.DMA for _ in range(NBUF)],
    ],
)
def _dispatch(x_hbm, packed_hbm, out_hbm, counts_hbm,
              pvb, cand, cidx, gch, rowbuf, cbuf, gsem, wsem):
    w = _wid()
    pno = w // 2
    h = w % 2
    pltpu.sync_copy(packed_hbm, pvb)

    iota = lax.iota(jnp.int32, L)

    def scan(i, cnt_s):
        v = pvb[pl.ds(i * L, L)]
        msk = (v & 15) == pno
        plsc.store_compressed(cand.at[pl.ds(cnt_s, L)], iota + i * L, mask=msk)
        pc = plsc.all_reduce_population_count(msk)
        if pc.ndim:
            pc = pc[0]
        return jnp.minimum(cnt_s + pc, C)

    cnt = lax.fori_loop(0, N // L, scan, jnp.zeros((), jnp.int32))

    @pl.when(h == 0)
    def _publish_count():
        cbuf[...] = jnp.broadcast_to(cnt, (L,))
        pltpu.sync_copy(cbuf, counts_hbm.at[pno])

    path_base = pno * C
    nocc = jnp.clip((cnt - h * CHUNK + 2 * CHUNK - 1) // (2 * CHUNK), 0, NCHUNK)

    def rank_of(c):
        return (2 * c + h) * CHUNK

    def fill_gather(c, b):
        rank0 = rank_of(c)
        for u in range(CHUNK // L):
            r = iota + (rank0 + u * L)
            valid = r < cnt
            ids = cand[pl.ds(rank0 + u * L, L)]
            ids = jnp.where(valid, ids, 0)
            pk = plsc.load_gather(pvb, [ids])
            g = lax.bitcast_convert_type(pk & -16, jnp.float32)
            g = jnp.where(valid, g, 0.0)
            cidx[b][pl.ds(u * L, L)] = ids
            gch[b][pl.ds(u * L, L)] = g
        pltpu.async_copy(x_hbm.at[cidx[b]], rowbuf[b], gsem[b])

    def scale_write(c, b):
        pltpu.make_async_copy(x_hbm.at[cidx[b]], rowbuf[b], gsem[b]).wait()

        def srow(j, carry2):
            gs = plsc.load_gather(gch[b], [jnp.zeros((L,), jnp.int32) + j])
            for k in range(D // L):
                rowbuf[b][j, pl.ds(k * L, L)] = (
                    rowbuf[b][j, pl.ds(k * L, L)] * gs)
            return carry2

        lax.fori_loop(0, CHUNK, srow, 0)
        pltpu.async_copy(
            rowbuf[b], out_hbm.at[pl.ds(path_base + rank_of(c), CHUNK)],
            wsem[b])

    @pl.when(nocc > 0)
    def _prime():
        fill_gather(0, 0)

    def group_body(grp, carry):
        for b in range(NBUF):
            c = grp * NBUF + b

            @pl.when(c < nocc)
            def _step(c=c, b=b):
                b1 = (b + 1) % NBUF
                cn = c + 1

                @pl.when(cn < nocc)
                def _lookahead():
                    @pl.when(cn >= NBUF)
                    def _reuse_wait():
                        pltpu.make_async_copy(
                            rowbuf[b1],
                            out_hbm.at[pl.ds(path_base, CHUNK)],
                            wsem[b1]).wait()

                    fill_gather(cn, b1)

                scale_write(c, b)

        return carry

    lax.fori_loop(0, (NCHUNK + NBUF - 1) // NBUF, group_body, 0)

    for b in range(NBUF):
        used = jnp.zeros((), jnp.bool_)
        for k in range(1, NBUF + 1):
            used = used | ((nocc >= k) & (lax.rem(nocc - k, NBUF) == b))

        @pl.when(used)
        def _drain(b=b):
            pltpu.make_async_copy(
                rowbuf[b], out_hbm.at[pl.ds(path_base, CHUNK)],
                wsem[b]).wait()


ZBIG = 512
ZSM = CHUNK


@functools.partial(
    pl.kernel,
    out_type=(),
    mesh=pltpu.create_tensorcore_mesh("core", num_cores=1),
    scratch_types=[
        pltpu.VMEM((ZBIG, D), jnp.float32),
        pltpu.SMEM((P, L), jnp.int32),
        pltpu.SemaphoreType.DMA,
        pltpu.SemaphoreType.DMA,
    ],
)
def _tc_zero(counts_hbm, out_hbm, zbuf, csm, sem_b, sem_s):
    pltpu.sync_copy(counts_hbm, csm)
    zbuf[...] = jnp.zeros((ZBIG, D), jnp.float32)
    nb_total = jnp.zeros((), jnp.int32)
    ns_total = jnp.zeros((), jnp.int32)
    for p in range(P):
        cnt = csm[p, 0]
        zs = ((cnt + CHUNK - 1) // CHUNK) * CHUNK
        za = jnp.minimum(((zs + ZBIG - 1) // ZBIG) * ZBIG, C)
        ns = (za - zs) // ZSM
        nb = (C - za) // ZBIG
        base = p * C

        def sfire(i, carry, base=base, zs=zs):
            pltpu.async_copy(
                zbuf.at[pl.ds(0, ZSM)],
                out_hbm.at[pl.ds(base + zs + i * ZSM, ZSM)], sem_s)
            return carry

        lax.fori_loop(0, ns, sfire, 0)

        def bfire(i, carry, base=base, za=za):
            pltpu.async_copy(
                zbuf, out_hbm.at[pl.ds(base + za + i * ZBIG, ZBIG)], sem_b)
            return carry

        lax.fori_loop(0, nb, bfire, 0)
        nb_total = nb_total + nb
        ns_total = ns_total + ns

    def sdrain(i, carry):
        pltpu.make_async_copy(
            zbuf.at[pl.ds(0, ZSM)], out_hbm.at[pl.ds(0, ZSM)], sem_s).wait()
        return carry

    lax.fori_loop(0, ns_total, sdrain, 0)

    def bdrain(i, carry):
        pltpu.make_async_copy(
            zbuf, out_hbm.at[pl.ds(0, ZBIG)], sem_b).wait()
        return carry

    lax.fori_loop(0, nb_total, bdrain, 0)


def kernel(x, scores):
    packed = _route(scores)
    out_ref = jax.new_ref(lax.empty((PC, D), jnp.float32))
    counts_ref = jax.new_ref(lax.empty((P, L), jnp.int32))
    _dispatch(x, packed, out_ref, counts_ref)
    _tc_zero(counts_ref, out_ref)
    return out_ref[...]


# speedup vs baseline: 1.1290x; 1.0117x over previous
import functools

import jax
import jax.numpy as jnp
from jax import lax
from jax.experimental import pallas as pl
from jax.experimental.pallas import tpu as pltpu
from jax.experimental.pallas import tpu_sc as plsc

N = 16384
D = 768
P = 16
C = 2048
PC = P * C
NC = 2
NS = 16
NW = NC * NS
L = 16

TOK_W = N // NW
ROWS_W = PC // NW
HALF = ROWS_W
CHUNK = 32
NCHUNK = ROWS_W // CHUNK
NBUF = 4
ZROWS = 16

_mesh = plsc.VectorSubcoreMesh(core_axis_name="c", subcore_axis_name="s")
_params = pltpu.CompilerParams(needs_layout_passes=False)


def _wid():
    return lax.axis_index("s") * NC + lax.axis_index("c")


def _scalar(a):
    return jnp.max(a) if a.ndim else a


@functools.partial(
    pl.kernel,
    out_type=jax.ShapeDtypeStruct((N,), jnp.int32),
    mesh=_mesh,
    compiler_params=_params,
    scratch_types=[
        pltpu.VMEM((TOK_W, P), jnp.float32),
        pltpu.VMEM((TOK_W,), jnp.int32),
    ],
)
def _route(scores_hbm, packed_hbm, sbuf, obuf):
    base = _wid() * TOK_W
    pltpu.sync_copy(scores_hbm.at[pl.ds(base, TOK_W)], sbuf)
    iota = lax.iota(jnp.int32, L)

    def body(t0, carry):
        rows = iota + t0 * L
        m = plsc.load_gather(sbuf, [rows, jnp.zeros((L,), jnp.int32)])
        am = jnp.zeros((L,), jnp.int32)
        for p in range(1, P):
            v = plsc.load_gather(sbuf, [rows, jnp.full((L,), p, jnp.int32)])
            gt = v > m
            m = jnp.where(gt, v, m)
            am = jnp.where(gt, p, am)
        packed = (lax.bitcast_convert_type(m, jnp.int32) & -16) | am
        obuf[pl.ds(t0 * L, L)] = packed
        return carry

    lax.fori_loop(0, TOK_W // L, body, 0)
    pltpu.sync_copy(obuf, packed_hbm.at[pl.ds(base, TOK_W)])


@functools.partial(
    pl.kernel,
    out_type=(),
    mesh=_mesh,
    compiler_params=_params,
    scratch_types=[
        pltpu.VMEM((N,), jnp.int32),
        pltpu.VMEM((C + L,), jnp.int32),
        [pltpu.VMEM((CHUNK,), jnp.int32) for _ in range(NBUF)],
        [pltpu.VMEM((CHUNK,), jnp.float32) for _ in range(NBUF)],
        [pltpu.VMEM((CHUNK, D), jnp.float32) for _ in range(NBUF)],
        pltpu.VMEM((L,), jnp.int32),
        [pltpu.SemaphoreType.DMA for _ in range(NBUF)],
        [pltpu.SemaphoreType.DMA for _ in range(NBUF)],
    ],
)
def _dispatch(x_hbm, packed_hbm, out_hbm, counts_hbm,
              pvb, cand, cidx, gch, rowbuf, cbuf, gsem, wsem):
    w = _wid()
    pno = w // 2
    h = w % 2
    pltpu.sync_copy(packed_hbm, pvb)

    iota = lax.iota(jnp.int32, L)

    def scan(i, cnt_s):
        v = pvb[pl.ds(i * L, L)]
        msk = (v & 15) == pno
        plsc.store_compressed(cand.at[pl.ds(cnt_s, L)], iota + i * L, mask=msk)
        pc = plsc.all_reduce_population_count(msk)
        if pc.ndim:
            pc = pc[0]
        return jnp.minimum(cnt_s + pc, C)

    cnt = lax.fori_loop(0, N // L, scan, jnp.zeros((), jnp.int32))

    @pl.when(h == 0)
    def _publish_count():
        cbuf[...] = jnp.broadcast_to(cnt, (L,))
        pltpu.sync_copy(cbuf, counts_hbm.at[pno])

    path_base = pno * C
    nocc = jnp.clip((cnt - h * CHUNK + 2 * CHUNK - 1) // (2 * CHUNK), 0, NCHUNK)

    def rank_of(c):
        return (2 * c + h) * CHUNK

    def fill_gather(c, b):
        rank0 = rank_of(c)
        for u in range(CHUNK // L):
            r = iota + (rank0 + u * L)
            valid = r < cnt
            ids = cand[pl.ds(rank0 + u * L, L)]
            ids = jnp.where(valid, ids, 0)
            pk = plsc.load_gather(pvb, [ids])
            g = lax.bitcast_convert_type(pk & -16, jnp.float32)
            g = jnp.where(valid, g, 0.0)
            cidx[b][pl.ds(u * L, L)] = ids
            gch[b][pl.ds(u * L, L)] = g
        pltpu.async_copy(x_hbm.at[cidx[b]], rowbuf[b], gsem[b])

    def scale_write(c, b):
        pltpu.make_async_copy(x_hbm.at[cidx[b]], rowbuf[b], gsem[b]).wait()

        def srow(j, carry2):
            gs = plsc.load_gather(gch[b], [jnp.zeros((L,), jnp.int32) + j])
            for k in range(D // L):
                rowbuf[b][j, pl.ds(k * L, L)] = (
                    rowbuf[b][j, pl.ds(k * L, L)] * gs)
            return carry2

        lax.fori_loop(0, CHUNK, srow, 0)
        pltpu.async_copy(
            rowbuf[b], out_hbm.at[pl.ds(path_base + rank_of(c), CHUNK)],
            wsem[b])

    for cp in range(2):
        @pl.when(cp < nocc)
        def _prime(cp=cp):
            fill_gather(cp, cp)

    def group_body(grp, carry):
        for b in range(NBUF):
            c = grp * NBUF + b

            @pl.when(c < nocc)
            def _step(c=c, b=b):
                b2 = (b + 2) % NBUF
                cn = c + 2

                @pl.when(cn < nocc)
                def _lookahead():
                    @pl.when(cn >= NBUF)
                    def _reuse_wait():
                        pltpu.make_async_copy(
                            rowbuf[b2],
                            out_hbm.at[pl.ds(path_base, CHUNK)],
                            wsem[b2]).wait()

                    fill_gather(cn, b2)

                scale_write(c, b)

        return carry

    lax.fori_loop(0, (NCHUNK + NBUF - 1) // NBUF, group_body, 0)

    for b in range(NBUF):
        used = jnp.zeros((), jnp.bool_)
        for k in range(1, NBUF + 1):
            used = used | ((nocc >= k) & (lax.rem(nocc - k, NBUF) == b))

        @pl.when(used)
        def _drain(b=b):
            pltpu.make_async_copy(
                rowbuf[b], out_hbm.at[pl.ds(path_base, CHUNK)],
                wsem[b]).wait()


ZTIERS = (CHUNK, 128, 512)


@functools.partial(
    pl.kernel,
    out_type=(),
    mesh=pltpu.create_tensorcore_mesh("core", num_cores=1),
    scratch_types=[
        pltpu.VMEM((ZTIERS[-1], D), jnp.float32),
        pltpu.SMEM((P, L), jnp.int32),
        [pltpu.SemaphoreType.DMA for _ in ZTIERS],
    ],
)
def _tc_zero(counts_hbm, out_hbm, zbuf, csm, sems):
    pltpu.sync_copy(counts_hbm, csm)
    zbuf[...] = jnp.zeros((ZTIERS[-1], D), jnp.float32)
    totals = [jnp.zeros((), jnp.int32) for _ in ZTIERS]
    for p in range(P):
        cnt = csm[p, 0]
        base = p * C
        cur = ((cnt + CHUNK - 1) // CHUNK) * CHUNK
        for t, rows in enumerate(ZTIERS):
            if t + 1 < len(ZTIERS):
                nxt = jnp.minimum(
                    ((cur + ZTIERS[t + 1] - 1) // ZTIERS[t + 1])
                    * ZTIERS[t + 1], C)
            else:
                nxt = C
            n = (nxt - cur) // rows

            def fire(i, carry, base=base, cur=cur, rows=rows, t=t):
                pltpu.async_copy(
                    zbuf.at[pl.ds(0, rows)],
                    out_hbm.at[pl.ds(base + cur + i * rows, rows)], sems[t])
                return carry

            lax.fori_loop(0, n, fire, 0)
            totals[t] = totals[t] + n
            cur = nxt

    for t, rows in enumerate(ZTIERS):
        def drain(i, carry, rows=rows, t=t):
            pltpu.make_async_copy(
                zbuf.at[pl.ds(0, rows)],
                out_hbm.at[pl.ds(0, rows)], sems[t]).wait()
            return carry

        lax.fori_loop(0, totals[t], drain, 0)


def kernel(x, scores):
    packed = _route(scores)
    out_ref = jax.new_ref(lax.empty((PC, D), jnp.float32))
    counts_ref = jax.new_ref(lax.empty((P, L), jnp.int32))
    _dispatch(x, packed, out_ref, counts_ref)
    _tc_zero(counts_ref, out_ref)
    return out_ref[...]
